# Initial kernel scaffold; baseline (speedup 1.0000x reference)
import functools

import jax
import jax.numpy as jnp
from jax import lax
from jax.experimental import pallas as pl
from jax.experimental.pallas import tpu as pltpu
from jax.experimental.pallas import tpu_sc as plsc

_B = 16384
_NS = 16
_LN = 16
_CH = 128


@functools.lru_cache(maxsize=None)
def _make_kernel(T, V, D, B):
    per_tile = T // _NS
    nch = per_tile // _CH
    rows_out = B // _NS

    mesh = plsc.VectorSubcoreMesh(core_axis_name="c", subcore_axis_name="s")

    @functools.partial(
        pl.kernel,
        out_type=jax.ShapeDtypeStruct((B, 2 * D), jnp.float32),
        mesh=mesh,
        scratch_types=[
            pltpu.VMEM((nch, _CH), jnp.int32),
            pltpu.VMEM((nch, _CH), jnp.int32),
            pltpu.VMEM((_CH, D), jnp.float32),
            pltpu.VMEM((_CH, D), jnp.float32),
            pltpu.VMEM_SHARED((B, D), jnp.float32),
            pltpu.SemaphoreType.DMA,
        ],
    )
    def k(pv, ps, uv, us, pt, ut, out, idx_v, seg_v, rows_v, zero_v, acc, sem):
        c = lax.axis_index("c")
        s = lax.axis_index("s")

        def zfill(i, _):
            zero_v[i // (D // _LN), pl.ds((i % (D // _LN)) * _LN, _LN)] = (
                jnp.zeros((_LN,), jnp.float32))
            return 0
        lax.fori_loop(0, _CH * D // _LN, zfill, 0)

        def zacc(i, _):
            pltpu.sync_copy(zero_v, acc.at[pl.ds(s * rows_out + i * _CH, _CH)])
            return 0
        lax.fori_loop(0, rows_out // _CH, zacc, 0)
        plsc.subcore_barrier()

        def feature(vals, segs, table):
            base = s * nch
            pltpu.sync_copy(vals.at[pl.ds(base, nch)], idx_v)
            pltpu.sync_copy(segs.at[pl.ds(base, nch)], seg_v)

            def body(j, _):
                pltpu.async_copy(table.at[idx_v.at[j]], rows_v, sem).wait()
                pltpu.sync_copy(rows_v, acc.at[seg_v.at[j]], add=True)
                return 0
            lax.fori_loop(0, nch, body, 0)

        @pl.when(c == 0)
        def _():
            feature(pv, ps, pt)

        @pl.when(c == 1)
        def _():
            feature(uv, us, ut)

        plsc.subcore_barrier()
        pltpu.sync_copy(
            acc.at[pl.ds(s * rows_out, rows_out)],
            out.at[pl.ds(s * rows_out, rows_out), pl.ds(c * D, D)])

    return k


def kernel(product_values, product_segment_ids, user_values, user_segment_ids,
           product_table, user_table):
    (T,) = product_values.shape
    V, D = product_table.shape
    k = _make_kernel(T, V, D, _B)
    n2 = T // _CH
    return k(product_values.reshape(n2, _CH),
             product_segment_ids.reshape(n2, _CH),
             user_values.reshape(n2, _CH),
             user_segment_ids.reshape(n2, _CH),
             product_table, user_table)


# speedup vs baseline: 14.7830x; 14.7830x over previous
import functools

import jax
import jax.numpy as jnp
from jax import lax
from jax.experimental import pallas as pl
from jax.experimental.pallas import tpu as pltpu
from jax.experimental.pallas import tpu_sc as plsc

_B = 16384
_NS = 16
_LN = 16
_CH = 128


@functools.lru_cache(maxsize=None)
def _make_kernel(T, V, D, B):
    per_tile = T // _NS
    nch = per_tile // _CH
    rows_out = B // _NS

    mesh = plsc.VectorSubcoreMesh(core_axis_name="c", subcore_axis_name="s")

    @functools.partial(
        pl.kernel,
        out_type=jax.ShapeDtypeStruct((B, 2 * D), jnp.float32),
        mesh=mesh,
        scratch_types=[
            pltpu.VMEM((nch, _CH), jnp.int32),
            pltpu.VMEM((nch, _CH), jnp.int32),
            pltpu.VMEM((_CH, D), jnp.float32),
            pltpu.VMEM((_CH, D), jnp.float32),
            pltpu.VMEM_SHARED((B, D), jnp.float32),
            pltpu.SemaphoreType.DMA,
        ],
        compiler_params=pltpu.CompilerParams(use_tc_tiling_on_sc=False),
    )
    def k(pv, ps, uv, us, pt, ut, out, idx_v, seg_v, rows_v, zero_v, acc, sem):
        c = lax.axis_index("c")
        s = lax.axis_index("s")

        def zfill(i, _):
            zero_v[i // (D // _LN), pl.ds((i % (D // _LN)) * _LN, _LN)] = (
                jnp.zeros((_LN,), jnp.float32))
            return 0
        lax.fori_loop(0, _CH * D // _LN, zfill, 0)

        def zacc(i, _):
            pltpu.sync_copy(zero_v, acc.at[pl.ds(s * rows_out + i * _CH, _CH)])
            return 0
        lax.fori_loop(0, rows_out // _CH, zacc, 0)
        plsc.subcore_barrier()

        def feature(vals, segs, table):
            base = s * nch
            pltpu.sync_copy(vals.at[pl.ds(base, nch)], idx_v)
            pltpu.sync_copy(segs.at[pl.ds(base, nch)], seg_v)

            def body(j, _):
                pltpu.async_copy(table.at[idx_v.at[j]], rows_v, sem).wait()
                pltpu.sync_copy(rows_v, acc.at[seg_v.at[j]], add=True)
                return 0
            lax.fori_loop(0, nch, body, 0)

        @pl.when(c == 0)
        def _():
            feature(pv, ps, pt)

        @pl.when(c == 1)
        def _():
            feature(uv, us, ut)

        plsc.subcore_barrier()
        pltpu.sync_copy(
            acc.at[pl.ds(s * rows_out, rows_out)],
            out.at[pl.ds(s * rows_out, rows_out), pl.ds(c * D, D)])

    return k


def kernel(product_values, product_segment_ids, user_values, user_segment_ids,
           product_table, user_table):
    (T,) = product_values.shape
    V, D = product_table.shape
    k = _make_kernel(T, V, D, _B)
    n2 = T // _CH
    return k(product_values.reshape(n2, _CH),
             product_segment_ids.reshape(n2, _CH),
             user_values.reshape(n2, _CH),
             user_segment_ids.reshape(n2, _CH),
             product_table, user_table)


# speedup vs baseline: 24.3408x; 1.6465x over previous
import functools

import jax
import jax.numpy as jnp
from jax import lax
from jax.experimental import pallas as pl
from jax.experimental.pallas import tpu as pltpu
from jax.experimental.pallas import tpu_sc as plsc

_B = 16384
_NS = 16
_LN = 16
_CH = 128


@functools.lru_cache(maxsize=None)
def _make_kernel(T, V, D, B):
    per_tile = T // _NS
    nch = per_tile // _CH
    rows_out = B // _NS

    mesh = plsc.VectorSubcoreMesh(core_axis_name="c", subcore_axis_name="s")

    @functools.partial(
        pl.kernel,
        out_type=jax.ShapeDtypeStruct((B, 2 * D), jnp.float32),
        mesh=mesh,
        scratch_types=[
            pltpu.VMEM((nch, _CH), jnp.int32),
            pltpu.VMEM((nch, _CH), jnp.int32),
            pltpu.VMEM((_CH, D), jnp.float32),
            pltpu.VMEM((_CH, D), jnp.float32),
            pltpu.VMEM((_CH, D), jnp.float32),
            pltpu.VMEM_SHARED((B, D), jnp.float32),
            pltpu.SemaphoreType.DMA,
            pltpu.SemaphoreType.DMA,
        ],
        compiler_params=pltpu.CompilerParams(use_tc_tiling_on_sc=False),
    )
    def k(pv, ps, uv, us, pt, ut, out, idx_v, seg_v, rows0, rows1, zero_v,
          acc, sem0, sem1):
        c = lax.axis_index("c")
        s = lax.axis_index("s")

        def zfill(i, _):
            zero_v[i // (D // _LN), pl.ds((i % (D // _LN)) * _LN, _LN)] = (
                jnp.zeros((_LN,), jnp.float32))
            return 0
        lax.fori_loop(0, _CH * D // _LN, zfill, 0)

        def zacc(i, _):
            pltpu.sync_copy(zero_v, acc.at[pl.ds(s * rows_out + i * _CH, _CH)])
            return 0
        lax.fori_loop(0, rows_out // _CH, zacc, 0)
        plsc.subcore_barrier()

        def feature(vals, segs, table):
            base = s * nch
            pltpu.sync_copy(vals.at[pl.ds(base, nch)], idx_v)
            pltpu.sync_copy(segs.at[pl.ds(base, nch)], seg_v)
            bufs = ((rows0, sem0), (rows1, sem1))

            pltpu.async_copy(table.at[idx_v.at[0]], rows0, sem0)

            def body(i, _):
                for b in range(2):
                    j = i * 2 + b
                    rows_b, sem_b = bufs[b]
                    rows_n, sem_n = bufs[1 - b]

                    @pl.when(j + 1 < nch)
                    def _():
                        pltpu.async_copy(table.at[idx_v.at[j + 1]], rows_n,
                                         sem_n)
                    pltpu.make_async_copy(table.at[idx_v.at[j]], rows_b,
                                          sem_b).wait()
                    pltpu.sync_copy(rows_b, acc.at[seg_v.at[j]], add=True)
                return 0
            lax.fori_loop(0, nch // 2, body, 0)

        @pl.when(c == 0)
        def _():
            feature(pv, ps, pt)

        @pl.when(c == 1)
        def _():
            feature(uv, us, ut)

        plsc.subcore_barrier()
        pltpu.sync_copy(
            acc.at[pl.ds(s * rows_out, rows_out)],
            out.at[pl.ds(s * rows_out, rows_out), pl.ds(c * D, D)])

    return k


def kernel(product_values, product_segment_ids, user_values, user_segment_ids,
           product_table, user_table):
    (T,) = product_values.shape
    V, D = product_table.shape
    k = _make_kernel(T, V, D, _B)
    n2 = T // _CH
    return k(product_values.reshape(n2, _CH),
             product_segment_ids.reshape(n2, _CH),
             user_values.reshape(n2, _CH),
             user_segment_ids.reshape(n2, _CH),
             product_table, user_table)
